# Initial kernel scaffold; baseline (speedup 1.0000x reference)
import jax
import jax.numpy as jnp
from jax.experimental import pallas as pl


def kernel(feat_idx, edge_list, emb_table, W1, b1, W2, b2):
    raise NotImplementedError("write your pallas kernel here")


# speedup vs baseline: 16.0300x; 16.0300x over previous
import functools

import jax
import jax.numpy as jnp
from jax import lax
from jax.experimental import pallas as pl
from jax.experimental.pallas import tpu as pltpu
from jax.experimental.pallas import tpu_sc as plsc

N = 10000
D = 128
E = 320000
NC = 2
NS = 16
NW = NC * NS
EPW = E // NW
ECHUNK = 128
NFULL = EPW // ECHUNK
ETAIL = EPW - NFULL * ECHUNK
RCH = 128
NRC = N // RCH
RTAIL = N - NRC * RCH
GCHUNK = 80
GN = N // GCHUNK

_mesh = plsc.VectorSubcoreMesh(core_axis_name="c", subcore_axis_name="s")


def _worker_ids():
    cid = lax.axis_index("c")
    sid = lax.axis_index("s")
    return cid, sid, cid * NS + sid


def _fill_rows(ref, nrows, ncols16, value):
    vec = jnp.full((16,), value, jnp.float32)

    def body(j, carry):
        for c in range(ncols16):
            ref[j, pl.ds(c * 16, 16)] = vec
        return carry

    lax.fori_loop(0, nrows, body, None)


def _rows_foreach(sid, fn_full, fn_tail):
    def body(i, carry):
        c = sid + i * NS

        @pl.when(c < NRC)
        def _():
            fn_full(c * RCH)

        @pl.when(c == NRC)
        def _():
            fn_tail(NRC * RCH)

        return carry

    lax.fori_loop(0, (NRC + NS) // NS, body, None)


def _sc_prep_body(dst_hbm, fidx_hbm, emb_hbm, hist_hbm, e0_hbm,
                  zb, ones_v, dsti, dsti_t, fidx_v, grows, hist_s, sem):
    cid, sid, wid = _worker_ids()

    _fill_rows(ones_v, ECHUNK, 1, 1.0)
    _fill_rows(zb, RCH, 1, 0.0)

    _rows_foreach(
        sid,
        lambda off: pltpu.sync_copy(zb, hist_s.at[pl.ds(off, RCH)]),
        lambda off: pltpu.sync_copy(zb.at[pl.ds(0, RTAIL)],
                                    hist_s.at[pl.ds(off, RTAIL)]),
    )
    plsc.subcore_barrier()

    def gbody(i, carry):
        c = wid + i * NW

        @pl.when(c < GN)
        def _():
            off = c * GCHUNK
            pltpu.sync_copy(fidx_hbm.at[pl.ds(off, GCHUNK)], fidx_v)
            pltpu.async_copy(emb_hbm.at[fidx_v], grows, sem).wait()
            pltpu.sync_copy(grows, e0_hbm.at[pl.ds(off, GCHUNK)])

        return carry

    lax.fori_loop(0, (GN + NW - 1) // NW, gbody, None)

    base = wid * EPW

    def hbody(i, carry):
        off = base + i * ECHUNK
        pltpu.sync_copy(dst_hbm.at[pl.ds(off, ECHUNK)], dsti)
        pltpu.sync_copy(ones_v, hist_s.at[dsti], add=True)
        return carry

    lax.fori_loop(0, NFULL, hbody, None)
    pltpu.sync_copy(dst_hbm.at[pl.ds(base + NFULL * ECHUNK, ETAIL)], dsti_t)
    pltpu.sync_copy(ones_v.at[pl.ds(0, ETAIL)], hist_s.at[dsti_t], add=True)

    plsc.subcore_barrier()
    _rows_foreach(
        sid,
        lambda off: pltpu.sync_copy(hist_s.at[pl.ds(off, RCH)],
                                    hist_hbm.at[cid, pl.ds(off, RCH)]),
        lambda off: pltpu.sync_copy(hist_s.at[pl.ds(off, RTAIL)],
                                    hist_hbm.at[cid, pl.ds(off, RTAIL)]),
    )


_sc_prep = pl.kernel(
    _sc_prep_body,
    out_type=(
        jax.ShapeDtypeStruct((NC, N, 16), jnp.float32),
        jax.ShapeDtypeStruct((N, D), jnp.float32),
    ),
    mesh=_mesh,
    scratch_types=[
        pltpu.VMEM((RCH, 16), jnp.float32),
        pltpu.VMEM((ECHUNK, 16), jnp.float32),
        pltpu.VMEM((ECHUNK,), jnp.int32),
        pltpu.VMEM((ETAIL,), jnp.int32),
        pltpu.VMEM((GCHUNK,), jnp.int32),
        pltpu.VMEM((GCHUNK, D), jnp.float32),
        pltpu.VMEM_SHARED((N, 16), jnp.float32),
        pltpu.SemaphoreType.DMA,
    ],
)


def _sc_edge_body(src_hbm, dst_hbm, y_hbm, parts_hbm,
                  src_i, dst_i, src_t, dst_t, rows, rows_t, zrow, acc, sem):
    cid, sid, wid = _worker_ids()

    _fill_rows(zrow, RCH, D // 16, 0.0)
    _rows_foreach(
        sid,
        lambda off: pltpu.sync_copy(zrow, acc.at[pl.ds(off, RCH)]),
        lambda off: pltpu.sync_copy(zrow.at[pl.ds(0, RTAIL)],
                                    acc.at[pl.ds(off, RTAIL)]),
    )
    plsc.subcore_barrier()

    base = wid * EPW

    def ebody(i, carry):
        off = base + i * ECHUNK
        pltpu.sync_copy(src_hbm.at[pl.ds(off, ECHUNK)], src_i)
        pltpu.sync_copy(dst_hbm.at[pl.ds(off, ECHUNK)], dst_i)
        pltpu.async_copy(y_hbm.at[src_i], rows, sem).wait()
        pltpu.sync_copy(rows, acc.at[dst_i], add=True)
        return carry

    lax.fori_loop(0, NFULL, ebody, None)

    off = base + NFULL * ECHUNK
    pltpu.sync_copy(src_hbm.at[pl.ds(off, ETAIL)], src_t)
    pltpu.sync_copy(dst_hbm.at[pl.ds(off, ETAIL)], dst_t)
    pltpu.async_copy(y_hbm.at[src_t], rows_t, sem).wait()
    pltpu.sync_copy(rows_t, acc.at[dst_t], add=True)

    plsc.subcore_barrier()
    _rows_foreach(
        sid,
        lambda off: pltpu.sync_copy(acc.at[pl.ds(off, RCH)],
                                    parts_hbm.at[cid, pl.ds(off, RCH)]),
        lambda off: pltpu.sync_copy(acc.at[pl.ds(off, RTAIL)],
                                    parts_hbm.at[cid, pl.ds(off, RTAIL)]),
    )


_sc_edge = pl.kernel(
    _sc_edge_body,
    out_type=jax.ShapeDtypeStruct((NC, N, D), jnp.float32),
    mesh=_mesh,
    scratch_types=[
        pltpu.VMEM((ECHUNK,), jnp.int32),
        pltpu.VMEM((ECHUNK,), jnp.int32),
        pltpu.VMEM((ETAIL,), jnp.int32),
        pltpu.VMEM((ETAIL,), jnp.int32),
        pltpu.VMEM((ECHUNK, D), jnp.float32),
        pltpu.VMEM((ETAIL, D), jnp.float32),
        pltpu.VMEM((RCH, D), jnp.float32),
        pltpu.VMEM_SHARED((N, D), jnp.float32),
        pltpu.SemaphoreType.DMA,
    ],
)


RB = 2000
GRID = N // RB


def _tc_a_body(deg_ref, e0_ref, w1_ref, out_ref):
    z = jnp.dot(e0_ref[...], w1_ref[...],
                preferred_element_type=jnp.float32,
                precision=lax.Precision.HIGHEST)
    out_ref[...] = lax.rsqrt(deg_ref[...]) * z


def _tc_b_body(deg_ref, parts_ref, y1_ref, b1_ref, w2_ref, out_ref):
    dinv = lax.rsqrt(deg_ref[...])
    pr = parts_ref[...]
    s = pr[0] + pr[1] + y1_ref[...]
    t = jnp.maximum(dinv * s + b1_ref[...], 0.0)
    z2 = jnp.dot(t, w2_ref[...],
                 preferred_element_type=jnp.float32,
                 precision=lax.Precision.HIGHEST)
    out_ref[...] = dinv * z2


def _tc_c_body(deg_ref, parts_ref, y2_ref, b2_ref, out_ref):
    dinv = lax.rsqrt(deg_ref[...])
    pr = parts_ref[...]
    s = pr[0] + pr[1] + y2_ref[...]
    out_ref[...] = dinv * s + b2_ref[...]


_parts_spec = pl.BlockSpec((NC, RB, D), lambda i: (0, i, 0))
_row_spec = pl.BlockSpec((RB, D), lambda i: (i, 0))
_w_spec = pl.BlockSpec((D, D), lambda i: (0, 0))
_b_spec = pl.BlockSpec((1, D), lambda i: (0, 0))
_out_sds = jax.ShapeDtypeStruct((N, D), jnp.float32)

_tc_a = pl.pallas_call(
    _tc_a_body, grid=(GRID,),
    in_specs=[_row_spec, _row_spec, _w_spec],
    out_specs=_row_spec, out_shape=_out_sds,
)
_tc_b = pl.pallas_call(
    _tc_b_body, grid=(GRID,),
    in_specs=[_row_spec, _parts_spec, _row_spec, _b_spec, _w_spec],
    out_specs=_row_spec, out_shape=_out_sds,
)
_tc_c = pl.pallas_call(
    _tc_c_body, grid=(GRID,),
    in_specs=[_row_spec, _parts_spec, _row_spec, _b_spec],
    out_specs=_row_spec, out_shape=_out_sds,
)


@jax.jit
def kernel(feat_idx, edge_list, emb_table, W1, b1, W2, b2):
    feat_idx = feat_idx.astype(jnp.int32)
    src = edge_list[0].astype(jnp.int32)
    dst = edge_list[1].astype(jnp.int32)
    b1 = b1.reshape(1, D)
    b2 = b2.reshape(1, D)

    hist, e0 = _sc_prep(dst, feat_idx, emb_table)
    deg2 = jnp.broadcast_to(hist[0, :, 0:1] + hist[1, :, 0:1] + 1.0, (N, D))
    y1 = _tc_a(deg2, e0, W1)
    p1 = _sc_edge(src, dst, y1)
    y2 = _tc_b(deg2, p1, y1, b1, W2)
    p2 = _sc_edge(src, dst, y2)
    return _tc_c(deg2, p2, y2, b2)
